# baseline (device time: 59486 ns/iter reference)
import numpy as np
import jax
import jax.numpy as jnp
from jax import lax
from jax.experimental import pallas as pl
from jax.experimental.pallas import tpu as pltpu

N_DEV = 4
SQ = 1024
D = 1024
HQ = 8
DH = 128
CH = SQ // N_DEV
HD = D // 2
SCALE = 0.08838834764831843

_inv = 1.0 / (10000.0 ** (np.arange(0, DH, 2) / DH))
_pos = np.arange(SQ)[:, None] * _inv[None, :]
_COS = np.tile(np.repeat(np.cos(_pos), 2, axis=-1), (1, HQ)).astype(np.float32)
_SIN = np.tile(np.repeat(np.sin(_pos), 2, axis=-1), (1, HQ)).astype(np.float32)

F32 = jnp.float32
BF16 = jnp.bfloat16


def kernel(x, Wq, Wk, Wv, Wo):

    def body(x_ref, wq_ref, wk_ref, wv_ref, wo_ref, cos_ref, sin_ref,
             out_ref, k_ref, v_ref, pr_ref, pl_ref, ownr_ref, ownl_ref,
             rsr_ref, rsl_ref, agr_ref, agl_ref,
             rsr_send, rsr_recv, rsl_send, rsl_recv,
             agr_send, agr_recv, agl_send, agl_recv):
        my = lax.axis_index("i")
        left = lax.rem(my + (N_DEV - 1), N_DEV)
        right = lax.rem(my + 1, N_DEV)
        diag = lax.rem(my + 2, N_DEV)

        bar = pltpu.get_barrier_semaphore()
        for nbr in (left, right, diag):
            pl.semaphore_signal(bar, inc=1, device_id=(nbr,),
                                device_id_type=pl.DeviceIdType.MESH)
        pl.semaphore_wait(bar, 3)

        def rope(t, cosr, sinr):
            n = t.shape[1]
            even = (lax.broadcasted_iota(jnp.int32, t.shape, 1) % 2) == 0
            t_next = pltpu.roll(t, n - 1, 1)
            t_prev = pltpu.roll(t, 1, 1)
            return t * cosr + jnp.where(even, -t_next, t_prev) * sinr

        xm = x_ref[0]

        k_ref[...] = rope(jnp.dot(xm, wk_ref[...],
                                  preferred_element_type=F32),
                          cos_ref[...], sin_ref[...])
        v_ref[...] = jnp.dot(xm, wv_ref[...], preferred_element_type=F32)

        def ctx_chunk(rc):
            ro = rc * CH
            xq = x_ref[0, pl.ds(ro, CH), :]
            q = rope(jnp.dot(xq, wq_ref[...], preferred_element_type=F32),
                     cos_ref[pl.ds(ro, CH), :], sin_ref[pl.ds(ro, CH), :])
            q = q * SCALE
            parts = []
            for h in range(HQ):
                sl = pl.ds(h * DH, DH)
                s = lax.dot_general(q[:, h * DH:(h + 1) * DH], k_ref[:, sl],
                                    (((1,), (1,)), ((), ())),
                                    preferred_element_type=F32)
                w = jnp.exp(s)
                ctx = jnp.dot(w, v_ref[:, sl], preferred_element_type=F32)
                parts.append(ctx / jnp.sum(w, axis=-1, keepdims=True))
            return jnp.concatenate(parts, axis=1)

        def proj_r(ctx):
            return jnp.dot(ctx, wo_ref[:, :HD],
                           preferred_element_type=F32).astype(BF16)

        def proj_l(ctx):
            return jnp.dot(ctx, wo_ref[:, HD:],
                           preferred_element_type=F32).astype(BF16)

        def copy(src, dst, send, recv, slot, dev):
            return pltpu.make_async_remote_copy(
                src_ref=src, dst_ref=dst.at[slot],
                send_sem=send.at[slot], recv_sem=recv.at[slot],
                device_id=(dev,), device_id_type=pl.DeviceIdType.MESH)

        sends = []

        ctx0 = ctx_chunk(my)
        pr_ref[0] = proj_r(ctx0)
        pl_ref[0] = proj_l(ctx0)
        c = copy(pr_ref.at[0], rsr_ref, rsr_send, rsr_recv, 0, left)
        c.start(); sends.append(c)
        c = copy(pl_ref.at[0], rsl_ref, rsl_send, rsl_recv, 0, right)
        c.start(); sends.append(c)

        ctx1 = ctx_chunk(right)
        pr_ref[1] = proj_r(ctx1)
        pl_ref[1] = proj_l(ctx1)
        c = copy(pl_ref.at[1], rsl_ref, rsl_send, rsl_recv, 1, diag)
        c.start(); sends.append(c)

        ctx2 = ctx_chunk(left)
        pr_ref[2] = proj_r(ctx2)
        pl_ref[2] = proj_l(ctx2)
        c = copy(pr_ref.at[2], rsr_ref, rsr_send, rsr_recv, 1, diag)
        c.start(); sends.append(c)

        ctx3 = ctx_chunk(diag)
        pr_ref[3] = proj_r(ctx3)
        pl_ref[3] = proj_l(ctx3)
        c = copy(pr_ref.at[3], rsr_ref, rsr_send, rsr_recv, 2, right)
        c.start(); sends.append(c)
        c = copy(pl_ref.at[3], rsl_ref, rsl_send, rsl_recv, 2, left)
        c.start(); sends.append(c)

        def recv_wait(dst, recv, slot):
            pltpu.make_async_remote_copy(
                src_ref=dst.at[slot], dst_ref=dst.at[slot],
                send_sem=recv.at[slot], recv_sem=recv.at[slot],
                device_id=(my,), device_id_type=pl.DeviceIdType.MESH,
            ).wait_recv()

        for j in range(3):
            recv_wait(rsr_ref, rsr_recv, j)
        own_r = (pr_ref[1].astype(F32) + rsr_ref[0].astype(F32)
                 + rsr_ref[1].astype(F32) + rsr_ref[2].astype(F32))
        ownr_ref[...] = own_r.astype(BF16)
        c = copy(ownr_ref, agr_ref, agr_send, agr_recv, 1, right)
        c.start(); sends.append(c)
        c = copy(ownr_ref, agr_ref, agr_send, agr_recv, 0, left)
        c.start(); sends.append(c)
        c = copy(ownr_ref, agr_ref, agr_send, agr_recv, 2, diag)
        c.start(); sends.append(c)
        out_ref[0, pl.ds(right * CH, CH), :HD] = own_r

        for j in range(3):
            recv_wait(rsl_ref, rsl_recv, j)
        own_l = (pl_ref[2].astype(F32) + rsl_ref[0].astype(F32)
                 + rsl_ref[1].astype(F32) + rsl_ref[2].astype(F32))
        ownl_ref[...] = own_l.astype(BF16)
        c = copy(ownl_ref, agl_ref, agl_send, agl_recv, 1, right)
        c.start(); sends.append(c)
        c = copy(ownl_ref, agl_ref, agl_send, agl_recv, 0, left)
        c.start(); sends.append(c)
        c = copy(ownl_ref, agl_ref, agl_send, agl_recv, 2, diag)
        c.start(); sends.append(c)
        out_ref[0, pl.ds(left * CH, CH), HD:] = own_l

        recv_wait(agr_ref, agr_recv, 0)
        out_ref[0, pl.ds(diag * CH, CH), :HD] = agr_ref[0].astype(F32)
        recv_wait(agr_ref, agr_recv, 1)
        out_ref[0, pl.ds(my * CH, CH), :HD] = agr_ref[1].astype(F32)
        recv_wait(agr_ref, agr_recv, 2)
        out_ref[0, pl.ds(left * CH, CH), :HD] = agr_ref[2].astype(F32)
        recv_wait(agl_ref, agl_recv, 0)
        out_ref[0, pl.ds(my * CH, CH), HD:] = agl_ref[0].astype(F32)
        recv_wait(agl_ref, agl_recv, 1)
        out_ref[0, pl.ds(diag * CH, CH), HD:] = agl_ref[1].astype(F32)
        recv_wait(agl_ref, agl_recv, 2)
        out_ref[0, pl.ds(right * CH, CH), HD:] = agl_ref[2].astype(F32)

        for d in sends:
            d.wait_send()

    cos = jnp.asarray(_COS)
    sin = jnp.asarray(_SIN)
    dma3 = pltpu.SemaphoreType.DMA((3,))
    return pl.pallas_call(
        body,
        out_shape=jax.ShapeDtypeStruct((1, SQ, D), F32),
        in_specs=[pl.BlockSpec(memory_space=pltpu.VMEM)] * 7,
        out_specs=pl.BlockSpec(memory_space=pltpu.VMEM),
        scratch_shapes=[
            pltpu.VMEM((SQ, D), F32),
            pltpu.VMEM((SQ, D), F32),
            pltpu.VMEM((N_DEV, CH, HD), BF16),
            pltpu.VMEM((N_DEV, CH, HD), BF16),
            pltpu.VMEM((CH, HD), BF16),
            pltpu.VMEM((CH, HD), BF16),
            pltpu.VMEM((3, CH, HD), BF16),
            pltpu.VMEM((3, CH, HD), BF16),
            pltpu.VMEM((3, CH, HD), BF16),
            pltpu.VMEM((3, CH, HD), BF16),
            dma3, dma3, dma3, dma3,
            dma3, dma3, dma3, dma3,
        ],
        compiler_params=pltpu.CompilerParams(
            collective_id=0, vmem_limit_bytes=100 * 1024 * 1024),
    )(x, Wq, Wk, Wv, Wo, cos, sin)
